# baseline (speedup 1.0000x reference)
import jax
import jax.numpy as jnp
from jax.experimental import pallas as pl
from jax.experimental.pallas import tpu as pltpu

N_ROWS = 100000
W = 1024
D = 512
ROW_BLOCK = 5000


def _fused_kernel(pattern_ref, proj_ref, stored_ref, pred_any,
                  pred_ref, sim_ref, idx_ref,
                  cvec, hsum, best_val, best_idx, sem):
    i = pl.program_id(0)

    @pl.when(i == 0)
    def _():
        projected = jax.lax.dot_general(
            pattern_ref[...], proj_ref[...],
            (((1,), (0,)), ((), ())),
            preferred_element_type=jnp.float32,
        )
        h = (projected > 0).astype(jnp.float32)
        cvec[...] = 2.0 * h - 1.0
        hsum[0] = jnp.sum(h)

    scores = jax.lax.dot_general(
        stored_ref[...], cvec[...],
        (((1,), (1,)), ((), ())),
        preferred_element_type=jnp.float32,
    )
    m = jnp.max(scores)
    rows = jax.lax.broadcasted_iota(jnp.int32, (ROW_BLOCK, 1), 0)
    local = jnp.min(jnp.where(scores == m, rows, N_ROWS))
    gidx = i * ROW_BLOCK + local

    @pl.when((i == 0) | (m > best_val[0]))
    def _():
        best_val[0] = m
        best_idx[0] = gidx

    @pl.when(i == pl.num_programs(0) - 1)
    def _():
        sim_ref[0, 0] = (W - hsum[0] + best_val[0]) / W
        idx_ref[0, 0] = best_idx[0]
        pltpu.make_async_copy(
            pred_any.at[pl.ds(best_idx[0], 1), :], pred_ref, sem,
        ).start()
        pltpu.make_async_copy(
            pred_any.at[pl.ds(best_idx[0], 1), :], pred_ref, sem,
        ).wait()


def kernel(pattern, hash_projections, stored_hashes, predictions):
    nblk = N_ROWS // ROW_BLOCK
    prediction, best_sim, best_idx = pl.pallas_call(
        _fused_kernel,
        grid=(nblk,),
        out_shape=(
            jax.ShapeDtypeStruct((1, D), jnp.float32),
            jax.ShapeDtypeStruct((1, 1), jnp.float32),
            jax.ShapeDtypeStruct((1, 1), jnp.int32),
        ),
        in_specs=[
            pl.BlockSpec((1, D), lambda i: (0, 0)),
            pl.BlockSpec((D, W), lambda i: (0, 0)),
            pl.BlockSpec((ROW_BLOCK, W), lambda i: (i, 0)),
            pl.BlockSpec(memory_space=pl.ANY),
        ],
        out_specs=(
            pl.BlockSpec((1, D), lambda i: (0, 0)),
            pl.BlockSpec(memory_space=pltpu.SMEM),
            pl.BlockSpec(memory_space=pltpu.SMEM),
        ),
        scratch_shapes=[
            pltpu.VMEM((1, W), jnp.float32),
            pltpu.SMEM((1,), jnp.float32),
            pltpu.SMEM((1,), jnp.float32),
            pltpu.SMEM((1,), jnp.int32),
            pltpu.SemaphoreType.DMA,
        ],
    )(pattern.reshape(1, D), hash_projections, stored_hashes, predictions)

    return (prediction.reshape(D), best_sim.reshape(()), best_idx.reshape(()))


# speedup vs baseline: 1.0086x; 1.0086x over previous
import jax
import jax.numpy as jnp
from jax.experimental import pallas as pl
from jax.experimental.pallas import tpu as pltpu

N_ROWS = 100000
W = 1024
D = 512
ROW_BLOCK = 2000


def _fused_kernel(pattern_ref, proj_ref, stored_ref, pred_any,
                  pred_ref, sim_ref, idx_ref,
                  cvec, hsum, best_val, best_idx, sem):
    i = pl.program_id(0)

    @pl.when(i == 0)
    def _():
        projected = jax.lax.dot_general(
            pattern_ref[...], proj_ref[...],
            (((1,), (0,)), ((), ())),
            preferred_element_type=jnp.float32,
        )
        h = (projected > 0).astype(jnp.float32)
        cvec[...] = 2.0 * h - 1.0
        hsum[0] = jnp.sum(h)

    scores = jax.lax.dot_general(
        stored_ref[...], cvec[...],
        (((1,), (1,)), ((), ())),
        preferred_element_type=jnp.float32,
    )
    m = jnp.max(scores)
    rows = jax.lax.broadcasted_iota(jnp.int32, (ROW_BLOCK, 1), 0)
    local = jnp.min(jnp.where(scores == m, rows, N_ROWS))
    gidx = i * ROW_BLOCK + local

    @pl.when((i == 0) | (m > best_val[0]))
    def _():
        best_val[0] = m
        best_idx[0] = gidx

    @pl.when(i == pl.num_programs(0) - 1)
    def _():
        sim_ref[0, 0] = (W - hsum[0] + best_val[0]) / W
        idx_ref[0, 0] = best_idx[0]
        pltpu.make_async_copy(
            pred_any.at[pl.ds(best_idx[0], 1), :], pred_ref, sem,
        ).start()
        pltpu.make_async_copy(
            pred_any.at[pl.ds(best_idx[0], 1), :], pred_ref, sem,
        ).wait()


def kernel(pattern, hash_projections, stored_hashes, predictions):
    nblk = N_ROWS // ROW_BLOCK
    prediction, best_sim, best_idx = pl.pallas_call(
        _fused_kernel,
        grid=(nblk,),
        out_shape=(
            jax.ShapeDtypeStruct((1, D), jnp.float32),
            jax.ShapeDtypeStruct((1, 1), jnp.float32),
            jax.ShapeDtypeStruct((1, 1), jnp.int32),
        ),
        in_specs=[
            pl.BlockSpec((1, D), lambda i: (0, 0)),
            pl.BlockSpec((D, W), lambda i: (0, 0)),
            pl.BlockSpec((ROW_BLOCK, W), lambda i: (i, 0)),
            pl.BlockSpec(memory_space=pl.ANY),
        ],
        out_specs=(
            pl.BlockSpec((1, D), lambda i: (0, 0)),
            pl.BlockSpec(memory_space=pltpu.SMEM),
            pl.BlockSpec(memory_space=pltpu.SMEM),
        ),
        scratch_shapes=[
            pltpu.VMEM((1, W), jnp.float32),
            pltpu.SMEM((1,), jnp.float32),
            pltpu.SMEM((1,), jnp.float32),
            pltpu.SMEM((1,), jnp.int32),
            pltpu.SemaphoreType.DMA,
        ],
    )(pattern.reshape(1, D), hash_projections, stored_hashes, predictions)

    return (prediction.reshape(D), best_sim.reshape(()), best_idx.reshape(()))
